# baseline (speedup 1.0000x reference)
import functools
import math

import jax
import jax.numpy as jnp
from jax import lax
from jax.experimental import pallas as pl
from jax.experimental.pallas import tpu as pltpu
from jax.experimental.pallas import tpu_sc as plsc

EMB_DIM = 64
D_MODEL = 768
SCALE = math.sqrt(EMB_DIM)

NC = 2
NS = 16
NW = NC * NS

CHUNK = 128
L = 16


def _sc_gather(table, idx2d, b_per_w):
    n_real = b_per_w // CHUNK
    nbuf = 2
    n_iter = (n_real + nbuf - 1) // nbuf
    n_pad = nbuf * n_iter + nbuf
    mesh = plsc.VectorSubcoreMesh(core_axis_name="c", subcore_axis_name="s")

    @functools.partial(
        pl.kernel,
        mesh=mesh,
        out_type=jax.ShapeDtypeStruct(
            (NW * b_per_w // 2, 2 * EMB_DIM), jnp.float32
        ),
        scratch_types=[
            pltpu.VMEM((n_pad * CHUNK,), jnp.int32),
            pltpu.VMEM((CHUNK, 2 * EMB_DIM), jnp.float32),
            pltpu.VMEM((CHUNK, 2 * EMB_DIM), jnp.float32),
            pltpu.SemaphoreType.DMA,
            pltpu.SemaphoreType.DMA,
        ],
        compiler_params=pltpu.CompilerParams(use_tc_tiling_on_sc=False),
    )
    def gather_kernel(idx_hbm, table_hbm, out_hbm, idx_v, buf0, buf1,
                      sem0, sem1):
        bufs = (buf0, buf1)
        sems = (sem0, sem1)
        wid = lax.axis_index("s") * NC + lax.axis_index("c")
        base = wid * b_per_w
        pltpu.sync_copy(idx_hbm.at[wid], idx_v.at[pl.ds(0, b_per_w)])

        zero16i = jnp.zeros((L,), jnp.int32)

        def ipad(i, carry):
            idx_v[pl.ds(b_per_w + i * L, L)] = zero16i
            return carry

        lax.fori_loop(0, (n_pad * CHUNK - b_per_w) // L, ipad, 0)

        def start_gather(c, buf, sem):
            return pltpu.async_copy(
                table_hbm.at[idx_v.at[pl.ds(c * CHUNK, CHUNK)]], buf, sem
            )

        def finish(c, buf, sem):
            pltpu.make_async_copy(
                table_hbm.at[idx_v.at[pl.ds(c * CHUNK, CHUNK)]], buf, sem
            ).wait()

            @pl.when(c < n_real)
            def _():
                j0 = base + c * CHUNK
                blk = jax.lax.shift_right_logical(j0, 11)
                rem = j0 & 2047
                half = jax.lax.shift_right_logical(rem, 10)
                k0 = rem & 1023
                pltpu.sync_copy(
                    buf.at[:, pl.ds(0, EMB_DIM)],
                    out_hbm.at[
                        pl.ds(blk * 1024 + k0, CHUNK),
                        pl.ds(half * EMB_DIM, EMB_DIM),
                    ],
                )

        for b in range(nbuf):
            start_gather(b, bufs[b], sems[b])

        def body(i, carry):
            c0 = nbuf * i
            for b in range(nbuf):
                finish(c0 + b, bufs[b], sems[b])
                start_gather(c0 + b + nbuf, bufs[b], sems[b])
            return carry

        lax.fori_loop(0, n_iter, body, 0)
        for b in range(nbuf):
            pltpu.make_async_copy(
                table_hbm.at[
                    idx_v.at[pl.ds((nbuf * n_iter + b) * CHUNK, CHUNK)]
                ],
                bufs[b], sems[b],
            ).wait()

    return gather_kernel(idx2d, table)


def _mm_body(x_ref, w_ref, b_ref, o_ref):
    bm = x_ref.shape[0]
    y = jnp.dot(x_ref[...], w_ref[...], preferred_element_type=jnp.float32)
    y = y + b_ref[...]
    o_ref[pl.ds(0, bm), :] = y[:, :D_MODEL]
    o_ref[pl.ds(bm, bm), :] = y[:, D_MODEL:]


def _tc_project(x, w2, b2, block_m):
    n2, kdim = x.shape
    return pl.pallas_call(
        _mm_body,
        grid=(n2 // block_m,),
        in_specs=[
            pl.BlockSpec((block_m, kdim), lambda i: (i, 0)),
            pl.BlockSpec((kdim, 2 * D_MODEL), lambda i: (0, 0)),
            pl.BlockSpec((1, 2 * D_MODEL), lambda i: (0, 0)),
        ],
        out_specs=pl.BlockSpec((2 * block_m, D_MODEL), lambda i: (i, 0)),
        out_shape=jax.ShapeDtypeStruct((2 * n2, D_MODEL), jnp.float32),
    )(x, w2, b2)


def kernel(persona, emb_table, proj_w, proj_b):
    batch, hist = persona.shape
    n = batch * hist
    b_per_w = n // NW
    idx2d = persona.astype(jnp.int32).T.reshape(NW, b_per_w)
    table128 = jnp.pad(emb_table, ((0, 0), (0, EMB_DIM)))
    packed = _sc_gather(table128, idx2d, b_per_w)
    wt8 = jnp.transpose(proj_w) * SCALE
    zz = jnp.zeros_like(wt8)
    w2 = jnp.block([[wt8, zz], [zz, wt8]])
    b2 = jnp.concatenate([proj_b, proj_b]).reshape(1, 2 * D_MODEL)
    out2d = _tc_project(packed, w2, b2, 1024)
    return out2d.reshape(hist, batch, D_MODEL).transpose(1, 0, 2)


# speedup vs baseline: 1.2654x; 1.2654x over previous
import functools
import math

import jax
import jax.numpy as jnp
from jax import lax
from jax.experimental import pallas as pl
from jax.experimental.pallas import tpu as pltpu
from jax.experimental.pallas import tpu_sc as plsc

EMB_DIM = 64
D_MODEL = 768
SCALE = math.sqrt(EMB_DIM)

NC = 2
NS = 16
NW = NC * NS

CHUNK = 128
L = 16


def _sc_gather(table, idx2d, b_per_w):
    n_real = b_per_w // CHUNK
    nbuf = 2
    n_iter = (n_real + nbuf - 1) // nbuf
    n_pad = nbuf * n_iter + nbuf
    mesh = plsc.VectorSubcoreMesh(core_axis_name="c", subcore_axis_name="s")

    @functools.partial(
        pl.kernel,
        mesh=mesh,
        out_type=jax.ShapeDtypeStruct(
            (NW * b_per_w // 2, 2 * EMB_DIM), jnp.float32
        ),
        scratch_types=[
            pltpu.VMEM((n_pad * CHUNK,), jnp.int32),
            pltpu.VMEM((CHUNK, EMB_DIM), jnp.float32),
            pltpu.VMEM((CHUNK, EMB_DIM), jnp.float32),
            pltpu.SemaphoreType.DMA,
            pltpu.SemaphoreType.DMA,
        ],
        compiler_params=pltpu.CompilerParams(use_tc_tiling_on_sc=False),
    )
    def gather_kernel(idx_hbm, table_hbm, out_hbm, idx_v, buf0, buf1,
                      sem0, sem1):
        bufs = (buf0, buf1)
        sems = (sem0, sem1)
        wid = lax.axis_index("s") * NC + lax.axis_index("c")
        base = wid * b_per_w
        pltpu.sync_copy(idx_hbm.at[wid], idx_v.at[pl.ds(0, b_per_w)])

        zero16i = jnp.zeros((L,), jnp.int32)

        def ipad(i, carry):
            idx_v[pl.ds(b_per_w + i * L, L)] = zero16i
            return carry

        lax.fori_loop(0, (n_pad * CHUNK - b_per_w) // L, ipad, 0)

        def start_gather(c, buf, sem):
            return pltpu.async_copy(
                table_hbm.at[idx_v.at[pl.ds(c * CHUNK, CHUNK)]], buf, sem
            )

        def finish_store(c, buf):
            j0 = base + c * CHUNK
            blk = jax.lax.shift_right_logical(j0, 11)
            rem = j0 & 2047
            half = jax.lax.shift_right_logical(rem, 10)
            k0 = rem & 1023
            pltpu.sync_copy(
                buf,
                out_hbm.at[
                    pl.ds(blk * 1024 + k0, CHUNK),
                    pl.ds(half * EMB_DIM, EMB_DIM),
                ],
            )

        def body(c, carry):
            start_gather(c, buf0, sem0).wait()
            finish_store(c, buf0)
            return carry

        lax.fori_loop(0, n_real, body, 0)

    return gather_kernel(idx2d, table)


def _mm_body(x_ref, w_ref, b_ref, o_ref):
    bm = x_ref.shape[0]
    y = jnp.dot(x_ref[...], w_ref[...], preferred_element_type=jnp.float32)
    y = y + b_ref[...]
    o_ref[pl.ds(0, bm), :] = y[:, :D_MODEL]
    o_ref[pl.ds(bm, bm), :] = y[:, D_MODEL:]


def _tc_project(x, w2, b2, block_m):
    n2, kdim = x.shape
    return pl.pallas_call(
        _mm_body,
        grid=(n2 // block_m,),
        in_specs=[
            pl.BlockSpec((block_m, kdim), lambda i: (i, 0)),
            pl.BlockSpec((kdim, 2 * D_MODEL), lambda i: (0, 0)),
            pl.BlockSpec((1, 2 * D_MODEL), lambda i: (0, 0)),
        ],
        out_specs=pl.BlockSpec((2 * block_m, D_MODEL), lambda i: (i, 0)),
        out_shape=jax.ShapeDtypeStruct((2 * n2, D_MODEL), jnp.float32),
    )(x, w2, b2)


def kernel(persona, emb_table, proj_w, proj_b):
    batch, hist = persona.shape
    n = batch * hist
    b_per_w = n // NW
    idx2d = persona.astype(jnp.int32).T.reshape(NW, b_per_w)
    packed = _sc_gather(emb_table, idx2d, b_per_w)
    wt8 = jnp.transpose(proj_w) * SCALE
    zz = jnp.zeros_like(wt8)
    w2 = jnp.block([[wt8, zz], [zz, wt8]])
    b2 = jnp.concatenate([proj_b, proj_b]).reshape(1, 2 * D_MODEL)
    out2d = _tc_project(packed, w2, b2, 1024)
    return out2d.reshape(hist, batch, D_MODEL).transpose(1, 0, 2)
